# Initial kernel scaffold; baseline (speedup 1.0000x reference)
import jax
import jax.numpy as jnp
from jax.experimental import pallas as pl


def kernel(rays_o, rays_d, grid_id, grid, atoms):
    raise NotImplementedError("write your pallas kernel here")


# speedup vs baseline: 2.5006x; 2.5006x over previous
import functools

import jax
import jax.numpy as jnp
from jax import lax
from jax.experimental import pallas as pl
from jax.experimental.pallas import tpu as pltpu
from jax.experimental.pallas import tpu_sc as plsc

SH_DIM = 9
DATA_DIM = SH_DIM * 3 + 1
COARSE = 64
FINE = 2
RADIUS = 1.3
NUM_ATOMS = 32
BATCH = 256
COARSE_VOX = RADIUS * 2.0 / COARSE
FINE_VOX = COARSE_VOX / FINE
STEP = FINE_VOX / 2.0
NSAMP = COARSE * 3 * 2 * FINE
NREAL = NSAMP - 1
P = BATCH * NSAMP
NP8 = P * 8

_C0 = 0.28209479177387814
_C1 = 0.4886025119029199
_C2 = 1.0925484305920792
_C3 = 0.31539156525252005
_C4 = 0.5462742152960396


_PREP_RB = 32


def _prep_body(o_ref, d_ref, idx_ref, wc_ref):
    o = o_ref[...]
    d = d_ref[...]
    off_in = jnp.minimum((RADIUS - o) / d, (-RADIUS - o) / d)
    start = jnp.max(off_in, axis=1, keepdims=True)
    steps = lax.broadcasted_iota(jnp.int32, (1, NSAMP), 1).astype(jnp.float32)
    t = start + steps * STEP
    iA = []
    wA = []
    for a in range(3):
        pa = o[:, a:a + 1] + t * d[:, a:a + 1]
        pn = jnp.clip((pa + RADIUS) / (2.0 * RADIUS), 0.0, 1.0)
        xg = pn * COARSE - 0.5
        i0f = jnp.floor(xg)
        fr = xg - i0f
        i0 = i0f.astype(jnp.int32)
        iA.append((jnp.clip(i0, 0, COARSE - 1), jnp.clip(i0 + 1, 0, COARSE - 1)))
        wA.append((1.0 - fr, fr))
    c = 0
    for dx in (0, 1):
        for dy in (0, 1):
            for dz in (0, 1):
                flat = (iA[0][dx] * COARSE + iA[1][dy]) * COARSE + iA[2][dz]
                idx_ref[c] = flat
                wc_ref[c] = wA[0][dx] * wA[1][dy] * wA[2][dz]
                c += 1


def _prep_call(rays_o, rays_d):
    return pl.pallas_call(
        _prep_body,
        grid=(BATCH // _PREP_RB,),
        in_specs=[
            pl.BlockSpec((_PREP_RB, 3), lambda i: (i, 0)),
            pl.BlockSpec((_PREP_RB, 3), lambda i: (i, 0)),
        ],
        out_specs=[
            pl.BlockSpec((8, _PREP_RB, NSAMP), lambda i: (0, i, 0)),
            pl.BlockSpec((8, _PREP_RB, NSAMP), lambda i: (0, i, 0)),
        ],
        out_shape=[
            jax.ShapeDtypeStruct((8, BATCH, NSAMP), jnp.int32),
            jax.ShapeDtypeStruct((8, BATCH, NSAMP), jnp.float32),
        ],
    )(rays_o, rays_d)


_TILE = 128
_TPTS = _TILE // 8


def _sc_pool(idx_flat, wc_flat, table):
    info = plsc.get_sparse_core_info()
    nc, ns = info.num_cores, info.num_subcores
    nw = nc * ns
    rows_pw = NP8 // nw
    pts_pw = P // nw
    ntiles = rows_pw // _TILE
    mesh = plsc.VectorSubcoreMesh(core_axis_name="c", subcore_axis_name="s")

    @functools.partial(
        pl.kernel,
        mesh=mesh,
        compiler_params=pltpu.CompilerParams(use_tc_tiling_on_sc=False),
        out_type=jax.ShapeDtypeStruct((P, NUM_ATOMS), jnp.float32),
        scratch_types=[
            pltpu.VMEM((_TILE,), jnp.int32),
            pltpu.VMEM((_TILE,), jnp.float32),
            pltpu.VMEM((_TILE, NUM_ATOMS), jnp.float32),
            pltpu.VMEM((_TPTS, NUM_ATOMS), jnp.float32),
            pltpu.SemaphoreType.DMA,
        ],
    )
    def k(idx_hbm, wc_hbm, table_hbm, out_hbm, idx_v, wc_v, rows_v, code_v, sem):
        wid = lax.axis_index("s") * nc + lax.axis_index("c")
        rbase = wid * rows_pw
        pbase = wid * pts_pw

        def body(tt, carry):
            roff = rbase + tt * _TILE
            poff = pbase + tt * _TPTS
            pltpu.sync_copy(idx_hbm.at[pl.ds(roff, _TILE)], idx_v)
            pltpu.sync_copy(wc_hbm.at[pl.ds(roff, _TILE)], wc_v)
            pltpu.async_copy(table_hbm.at[idx_v], rows_v, sem).wait()
            for j in range(_TPTS // 2):
                wchunk = wc_v[pl.ds(16 * j, 16)]
                for pi in range(2):
                    i = 2 * j + pi
                    for h in range(NUM_ATOMS // 16):
                        acc = jnp.zeros((16,), jnp.float32)
                        for c in range(8):
                            w = wchunk[8 * pi + c]
                            acc = acc + w * rows_v[8 * i + c, pl.ds(16 * h, 16)]
                        code_v[i, pl.ds(16 * h, 16)] = acc
            pltpu.sync_copy(code_v, out_hbm.at[pl.ds(poff, _TPTS)])
            return carry

        lax.fori_loop(0, ntiles, body, 0)

    return k(idx_flat, wc_flat, table)


_REND_RB = 8


def _render_body(code_ref, o_ref, d_ref, m_ref, rgb_ref, alpha_ref, depth_ref):
    o = o_ref[...]
    d = d_ref[...]
    off_in = jnp.minimum((RADIUS - o) / d, (-RADIUS - o) / d)
    start = jnp.max(off_in, axis=1, keepdims=True)
    steps = lax.broadcasted_iota(jnp.int32, (1, NSAMP), 1).astype(jnp.float32)
    t = start + steps * STEP

    fine_w = []
    inb = None
    for a in range(3):
        pa = o[:, a:a + 1] + t * d[:, a:a + 1]
        in_a = (pa > -RADIUS) & (pa < RADIUS)
        inb = in_a if inb is None else (inb & in_a)
        pn = jnp.clip((pa + RADIUS) / (2.0 * RADIUS), 0.0, 1.0)
        xc = pn * COARSE
        local = xc - jnp.floor(xc)
        xf = local * FINE - 0.5
        j0f = jnp.floor(xf)
        ff = xf - j0f
        a1 = jnp.where(j0f < -0.5, 0.0, jnp.where(j0f > 0.5, 1.0, ff))
        fine_w.append((1.0 - a1, a1))

    code2 = code_ref[...]
    m = m_ref[...]
    data3 = jnp.zeros((_REND_RB, NSAMP, DATA_DIM), jnp.float32)
    f = 0
    for jx in (0, 1):
        for jy in (0, 1):
            for jz in (0, 1):
                wf = fine_w[0][jx] * fine_w[1][jy] * fine_w[2][jz]
                mf = m[f * NUM_ATOMS:(f + 1) * NUM_ATOMS, :]
                tf = jnp.dot(code2, mf, preferred_element_type=jnp.float32)
                data3 = data3 + wf[:, :, None] * tf.reshape(_REND_RB, NSAMP, DATA_DIM)
                f += 1

    svalid = lax.broadcasted_iota(jnp.int32, (1, NSAMP), 1) < NREAL
    mask = inb & svalid
    sig_raw = data3[:, :, DATA_DIM - 1]
    sigma = jnp.where(mask, jnp.maximum(sig_raw, 0.0), 0.0)
    dn = jnp.sqrt(jnp.sum(d * d, axis=1, keepdims=True))
    t_next = start + (steps + 1.0) * STEP
    deltas = (t_next - t) * dn
    x = sigma * deltas
    alpha = 1.0 - jnp.exp(-x)
    tau = jnp.log((1.0 - alpha) + 1e-10)
    ri = lax.broadcasted_iota(jnp.int32, (NSAMP, NSAMP), 0)
    ci = lax.broadcasted_iota(jnp.int32, (NSAMP, NSAMP), 1)
    lt = (ri < ci).astype(jnp.float32)
    texc = jnp.exp(jnp.dot(tau, lt, preferred_element_type=jnp.float32))
    w_al = alpha * texc
    acc_w = jnp.sum(w_al, axis=1, keepdims=True)

    dx_ = d[:, 0:1]
    dy_ = d[:, 1:2]
    dz_ = d[:, 2:3]
    sh = [
        jnp.full_like(dx_, _C0),
        -_C1 * dy_,
        _C1 * dz_,
        -_C1 * dx_,
        _C2 * dx_ * dy_,
        -_C2 * dy_ * dz_,
        _C3 * (2.0 * dz_ * dz_ - dx_ * dx_ - dy_ * dy_),
        -_C2 * dx_ * dz_,
        _C4 * (dx_ * dx_ - dy_ * dy_),
    ]
    outs = []
    for kk in range(3):
        rgb_k = jnp.zeros((_REND_RB, NSAMP), jnp.float32)
        for s in range(SH_DIM):
            rgb_k = rgb_k + data3[:, :, kk * SH_DIM + s] * sh[s]
        csig = 1.0 / (1.0 + jnp.exp(-rgb_k))
        outs.append(jnp.sum(w_al * csig, axis=1, keepdims=True) + 1.0 - acc_w)
    rgb_ref[...] = jnp.concatenate(outs, axis=1)
    alpha_ref[...] = alpha
    depth_ref[...] = jnp.sum(w_al * t, axis=1, keepdims=True)


def _render_call(code, rays_o, rays_d, m2):
    return pl.pallas_call(
        _render_body,
        grid=(BATCH // _REND_RB,),
        in_specs=[
            pl.BlockSpec((_REND_RB * NSAMP, NUM_ATOMS), lambda i: (i, 0)),
            pl.BlockSpec((_REND_RB, 3), lambda i: (i, 0)),
            pl.BlockSpec((_REND_RB, 3), lambda i: (i, 0)),
            pl.BlockSpec((8 * NUM_ATOMS, DATA_DIM), lambda i: (0, 0)),
        ],
        out_specs=[
            pl.BlockSpec((_REND_RB, 3), lambda i: (i, 0)),
            pl.BlockSpec((_REND_RB, NSAMP), lambda i: (i, 0)),
            pl.BlockSpec((_REND_RB, 1), lambda i: (i, 0)),
        ],
        out_shape=[
            jax.ShapeDtypeStruct((BATCH, 3), jnp.float32),
            jax.ShapeDtypeStruct((BATCH, NSAMP), jnp.float32),
            jax.ShapeDtypeStruct((BATCH, 1), jnp.float32),
        ],
    )(code, rays_o, rays_d, m2)


def kernel(rays_o, rays_d, grid_id, grid, atoms):
    del grid_id
    idx8, wc8 = _prep_call(rays_o, rays_d)
    idx_pm = jnp.transpose(idx8, (1, 2, 0)).reshape(-1)
    wc_pm = jnp.transpose(wc8, (1, 2, 0)).reshape(-1)
    code = _sc_pool(idx_pm, wc_pm, grid)
    m2 = atoms.reshape(8 * NUM_ATOMS, DATA_DIM)
    rgb, alpha_full, depth2 = _render_call(code, rays_o, rays_d, m2)
    return (rgb, alpha_full[:, :NREAL], depth2.reshape(BATCH))


# speedup vs baseline: 2.9731x; 1.1890x over previous
import functools

import jax
import jax.numpy as jnp
from jax import lax
from jax.experimental import pallas as pl
from jax.experimental.pallas import tpu as pltpu
from jax.experimental.pallas import tpu_sc as plsc

SH_DIM = 9
DATA_DIM = SH_DIM * 3 + 1
COARSE = 64
FINE = 2
RADIUS = 1.3
NUM_ATOMS = 32
BATCH = 256
COARSE_VOX = RADIUS * 2.0 / COARSE
FINE_VOX = COARSE_VOX / FINE
STEP = FINE_VOX / 2.0
NSAMP = COARSE * 3 * 2 * FINE
NREAL = NSAMP - 1
P = BATCH * NSAMP
NP8 = P * 8

_C0 = 0.28209479177387814
_C1 = 0.4886025119029199
_C2 = 1.0925484305920792
_C3 = 0.31539156525252005
_C4 = 0.5462742152960396


_PREP_RB = 32


def _prep_body(o_ref, d_ref, idx_ref, wc_ref):
    o = o_ref[...]
    d = d_ref[...]
    off_in = jnp.minimum((RADIUS - o) / d, (-RADIUS - o) / d)
    start = jnp.max(off_in, axis=1, keepdims=True)
    steps = lax.broadcasted_iota(jnp.int32, (1, NSAMP), 1).astype(jnp.float32)
    t = start + steps * STEP
    iA = []
    wA = []
    for a in range(3):
        pa = o[:, a:a + 1] + t * d[:, a:a + 1]
        pn = jnp.clip((pa + RADIUS) / (2.0 * RADIUS), 0.0, 1.0)
        xg = pn * COARSE - 0.5
        i0f = jnp.floor(xg)
        fr = xg - i0f
        i0 = i0f.astype(jnp.int32)
        iA.append((jnp.clip(i0, 0, COARSE - 1), jnp.clip(i0 + 1, 0, COARSE - 1)))
        wA.append((1.0 - fr, fr))
    c = 0
    for dx in (0, 1):
        for dy in (0, 1):
            for dz in (0, 1):
                flat = (iA[0][dx] * COARSE + iA[1][dy]) * COARSE + iA[2][dz]
                idx_ref[c] = flat
                wc_ref[c] = wA[0][dx] * wA[1][dy] * wA[2][dz]
                c += 1


def _prep_call(rays_o, rays_d):
    return pl.pallas_call(
        _prep_body,
        grid=(BATCH // _PREP_RB,),
        in_specs=[
            pl.BlockSpec((_PREP_RB, 3), lambda i: (i, 0)),
            pl.BlockSpec((_PREP_RB, 3), lambda i: (i, 0)),
        ],
        out_specs=[
            pl.BlockSpec((8, _PREP_RB, NSAMP), lambda i: (0, i, 0)),
            pl.BlockSpec((8, _PREP_RB, NSAMP), lambda i: (0, i, 0)),
        ],
        out_shape=[
            jax.ShapeDtypeStruct((8, BATCH, NSAMP), jnp.int32),
            jax.ShapeDtypeStruct((8, BATCH, NSAMP), jnp.float32),
        ],
    )(rays_o, rays_d)


_TILE = 128
_TPTS = _TILE // 8


_STILE = 4
_SROWS = _STILE * _TILE
_SPTS = _STILE * _TPTS


def _sc_pool(idx_flat, wc_flat, table):
    info = plsc.get_sparse_core_info()
    nc, ns = info.num_cores, info.num_subcores
    nw = nc * ns
    rows_pw = NP8 // nw
    pts_pw = P // nw
    nsup = rows_pw // _SROWS
    mesh = plsc.VectorSubcoreMesh(core_axis_name="c", subcore_axis_name="s")

    @functools.partial(
        pl.kernel,
        mesh=mesh,
        compiler_params=pltpu.CompilerParams(use_tc_tiling_on_sc=False),
        out_type=jax.ShapeDtypeStruct((P, NUM_ATOMS), jnp.float32),
        scratch_types=[
            pltpu.VMEM((2, _SROWS), jnp.int32),
            pltpu.VMEM((2, _SROWS), jnp.float32),
            pltpu.VMEM((2, _SROWS, NUM_ATOMS), jnp.float32),
            pltpu.VMEM((2, _SPTS, NUM_ATOMS), jnp.float32),
            pltpu.SemaphoreType.DMA,
            pltpu.SemaphoreType.DMA,
            pltpu.SemaphoreType.DMA,
        ],
    )
    def k(idx_hbm, wc_hbm, table_hbm, out_hbm, idx_v, wc_v, rows_v, code_v,
          gsem, ssem0, ssem1):
        wid = lax.axis_index("s") * nc + lax.axis_index("c")
        rbase = wid * rows_pw
        pbase = wid * pts_pw
        ssems = (ssem0, ssem1)

        def stage_and_fire(sup, par):
            roff = rbase + sup * _SROWS
            pltpu.sync_copy(idx_hbm.at[pl.ds(roff, _SROWS)], idx_v.at[par])
            pltpu.sync_copy(wc_hbm.at[pl.ds(roff, _SROWS)], wc_v.at[par])
            for j in range(_STILE):
                pltpu.async_copy(
                    table_hbm.at[idx_v.at[par, pl.ds(j * _TILE, _TILE)]],
                    rows_v.at[par, pl.ds(j * _TILE, _TILE)], gsem)

        def drain_gathers(par):
            for j in range(_STILE):
                pltpu.make_async_copy(
                    table_hbm.at[idx_v.at[par, pl.ds(j * _TILE, _TILE)]],
                    rows_v.at[par, pl.ds(j * _TILE, _TILE)], gsem).wait()

        def pool(par):
            for jj in range(_SPTS // 2):
                wchunk = wc_v[par, pl.ds(16 * jj, 16)]
                for pi in range(2):
                    i = 2 * jj + pi
                    ws = [wchunk[8 * pi + c] for c in range(8)]
                    for h in range(NUM_ATOMS // 16):
                        acc = ws[0] * rows_v[par, 8 * i, pl.ds(16 * h, 16)]
                        for c in range(1, 8):
                            acc = acc + ws[c] * rows_v[par, 8 * i + c,
                                                       pl.ds(16 * h, 16)]
                        code_v[par, i, pl.ds(16 * h, 16)] = acc

        def store_desc(sup, par):
            poff = pbase + sup * _SPTS
            return pltpu.make_async_copy(
                code_v.at[par], out_hbm.at[pl.ds(poff, _SPTS)], ssems[par])

        stage_and_fire(0, 0)

        def body(g, carry):
            for par in range(2):
                sup = 2 * g + par
                drain_gathers(par)
                if par == 0:
                    stage_and_fire(sup + 1, 1)
                else:
                    @pl.when(g < nsup // 2 - 1)
                    def _():
                        stage_and_fire(sup + 1, 0)
                @pl.when(g >= 1)
                def _():
                    store_desc(0, par).wait()
                pool(par)
                store_desc(sup, par).start()
            return carry

        lax.fori_loop(0, nsup // 2, body, 0)
        store_desc(0, 0).wait()
        store_desc(0, 1).wait()

    return k(idx_flat, wc_flat, table)


_REND_RB = 8


def _render_body(code_ref, o_ref, d_ref, m_ref, rgb_ref, alpha_ref, depth_ref):
    o = o_ref[...]
    d = d_ref[...]
    off_in = jnp.minimum((RADIUS - o) / d, (-RADIUS - o) / d)
    start = jnp.max(off_in, axis=1, keepdims=True)
    steps = lax.broadcasted_iota(jnp.int32, (1, NSAMP), 1).astype(jnp.float32)
    t = start + steps * STEP

    fine_w = []
    inb = None
    for a in range(3):
        pa = o[:, a:a + 1] + t * d[:, a:a + 1]
        in_a = (pa > -RADIUS) & (pa < RADIUS)
        inb = in_a if inb is None else (inb & in_a)
        pn = jnp.clip((pa + RADIUS) / (2.0 * RADIUS), 0.0, 1.0)
        xc = pn * COARSE
        local = xc - jnp.floor(xc)
        xf = local * FINE - 0.5
        j0f = jnp.floor(xf)
        ff = xf - j0f
        a1 = jnp.where(j0f < -0.5, 0.0, jnp.where(j0f > 0.5, 1.0, ff))
        fine_w.append((1.0 - a1, a1))

    code2 = code_ref[...]
    m = m_ref[...]
    data3 = jnp.zeros((_REND_RB, NSAMP, DATA_DIM), jnp.float32)
    f = 0
    for jx in (0, 1):
        for jy in (0, 1):
            for jz in (0, 1):
                wf = fine_w[0][jx] * fine_w[1][jy] * fine_w[2][jz]
                mf = m[f * NUM_ATOMS:(f + 1) * NUM_ATOMS, :]
                tf = jnp.dot(code2, mf, preferred_element_type=jnp.float32)
                data3 = data3 + wf[:, :, None] * tf.reshape(_REND_RB, NSAMP, DATA_DIM)
                f += 1

    svalid = lax.broadcasted_iota(jnp.int32, (1, NSAMP), 1) < NREAL
    mask = inb & svalid
    sig_raw = data3[:, :, DATA_DIM - 1]
    sigma = jnp.where(mask, jnp.maximum(sig_raw, 0.0), 0.0)
    dn = jnp.sqrt(jnp.sum(d * d, axis=1, keepdims=True))
    t_next = start + (steps + 1.0) * STEP
    deltas = (t_next - t) * dn
    x = sigma * deltas
    alpha = 1.0 - jnp.exp(-x)
    tau = jnp.log((1.0 - alpha) + 1e-10)
    ri = lax.broadcasted_iota(jnp.int32, (NSAMP, NSAMP), 0)
    ci = lax.broadcasted_iota(jnp.int32, (NSAMP, NSAMP), 1)
    lt = (ri < ci).astype(jnp.float32)
    texc = jnp.exp(jnp.dot(tau, lt, preferred_element_type=jnp.float32))
    w_al = alpha * texc
    acc_w = jnp.sum(w_al, axis=1, keepdims=True)

    dx_ = d[:, 0:1]
    dy_ = d[:, 1:2]
    dz_ = d[:, 2:3]
    sh = [
        jnp.full_like(dx_, _C0),
        -_C1 * dy_,
        _C1 * dz_,
        -_C1 * dx_,
        _C2 * dx_ * dy_,
        -_C2 * dy_ * dz_,
        _C3 * (2.0 * dz_ * dz_ - dx_ * dx_ - dy_ * dy_),
        -_C2 * dx_ * dz_,
        _C4 * (dx_ * dx_ - dy_ * dy_),
    ]
    outs = []
    for kk in range(3):
        rgb_k = jnp.zeros((_REND_RB, NSAMP), jnp.float32)
        for s in range(SH_DIM):
            rgb_k = rgb_k + data3[:, :, kk * SH_DIM + s] * sh[s]
        csig = 1.0 / (1.0 + jnp.exp(-rgb_k))
        outs.append(jnp.sum(w_al * csig, axis=1, keepdims=True) + 1.0 - acc_w)
    rgb_ref[...] = jnp.concatenate(outs, axis=1)
    alpha_ref[...] = alpha
    depth_ref[...] = jnp.sum(w_al * t, axis=1, keepdims=True)


def _render_call(code, rays_o, rays_d, m2):
    return pl.pallas_call(
        _render_body,
        grid=(BATCH // _REND_RB,),
        in_specs=[
            pl.BlockSpec((_REND_RB * NSAMP, NUM_ATOMS), lambda i: (i, 0)),
            pl.BlockSpec((_REND_RB, 3), lambda i: (i, 0)),
            pl.BlockSpec((_REND_RB, 3), lambda i: (i, 0)),
            pl.BlockSpec((8 * NUM_ATOMS, DATA_DIM), lambda i: (0, 0)),
        ],
        out_specs=[
            pl.BlockSpec((_REND_RB, 3), lambda i: (i, 0)),
            pl.BlockSpec((_REND_RB, NSAMP), lambda i: (i, 0)),
            pl.BlockSpec((_REND_RB, 1), lambda i: (i, 0)),
        ],
        out_shape=[
            jax.ShapeDtypeStruct((BATCH, 3), jnp.float32),
            jax.ShapeDtypeStruct((BATCH, NSAMP), jnp.float32),
            jax.ShapeDtypeStruct((BATCH, 1), jnp.float32),
        ],
    )(code, rays_o, rays_d, m2)


def kernel(rays_o, rays_d, grid_id, grid, atoms):
    del grid_id
    idx8, wc8 = _prep_call(rays_o, rays_d)
    idx_pm = jnp.transpose(idx8, (1, 2, 0)).reshape(-1)
    wc_pm = jnp.transpose(wc8, (1, 2, 0)).reshape(-1)
    code = _sc_pool(idx_pm, wc_pm, grid)
    m2 = atoms.reshape(8 * NUM_ATOMS, DATA_DIM)
    rgb, alpha_full, depth2 = _render_call(code, rays_o, rays_d, m2)
    return (rgb, alpha_full[:, :NREAL], depth2.reshape(BATCH))


# speedup vs baseline: 3.0607x; 1.0295x over previous
import functools

import jax
import jax.numpy as jnp
from jax import lax
from jax.experimental import pallas as pl
from jax.experimental.pallas import tpu as pltpu
from jax.experimental.pallas import tpu_sc as plsc

SH_DIM = 9
DATA_DIM = SH_DIM * 3 + 1
COARSE = 64
FINE = 2
RADIUS = 1.3
NUM_ATOMS = 32
BATCH = 256
COARSE_VOX = RADIUS * 2.0 / COARSE
FINE_VOX = COARSE_VOX / FINE
STEP = FINE_VOX / 2.0
NSAMP = COARSE * 3 * 2 * FINE
NREAL = NSAMP - 1
P = BATCH * NSAMP
NP8 = P * 8

_C0 = 0.28209479177387814
_C1 = 0.4886025119029199
_C2 = 1.0925484305920792
_C3 = 0.31539156525252005
_C4 = 0.5462742152960396


_PREP_RB = 32


def _prep_body(o_ref, d_ref, idx_ref, wc_ref):
    o = o_ref[...]
    d = d_ref[...]
    off_in = jnp.minimum((RADIUS - o) / d, (-RADIUS - o) / d)
    start = jnp.max(off_in, axis=1, keepdims=True)
    steps = lax.broadcasted_iota(jnp.int32, (1, NSAMP), 1).astype(jnp.float32)
    t = start + steps * STEP
    iA = []
    wA = []
    for a in range(3):
        pa = o[:, a:a + 1] + t * d[:, a:a + 1]
        pn = jnp.clip((pa + RADIUS) / (2.0 * RADIUS), 0.0, 1.0)
        xg = pn * COARSE - 0.5
        i0f = jnp.floor(xg)
        fr = xg - i0f
        i0 = i0f.astype(jnp.int32)
        iA.append((jnp.clip(i0, 0, COARSE - 1), jnp.clip(i0 + 1, 0, COARSE - 1)))
        wA.append((1.0 - fr, fr))
    c = 0
    for dx in (0, 1):
        for dy in (0, 1):
            for dz in (0, 1):
                flat = (iA[0][dx] * COARSE + iA[1][dy]) * COARSE + iA[2][dz]
                idx_ref[c] = flat
                wc_ref[c] = wA[0][dx] * wA[1][dy] * wA[2][dz]
                c += 1


def _prep_call(rays_o, rays_d):
    return pl.pallas_call(
        _prep_body,
        grid=(BATCH // _PREP_RB,),
        in_specs=[
            pl.BlockSpec((_PREP_RB, 3), lambda i: (i, 0)),
            pl.BlockSpec((_PREP_RB, 3), lambda i: (i, 0)),
        ],
        out_specs=[
            pl.BlockSpec((8, _PREP_RB, NSAMP), lambda i: (0, i, 0)),
            pl.BlockSpec((8, _PREP_RB, NSAMP), lambda i: (0, i, 0)),
        ],
        out_shape=[
            jax.ShapeDtypeStruct((8, BATCH, NSAMP), jnp.int32),
            jax.ShapeDtypeStruct((8, BATCH, NSAMP), jnp.float32),
        ],
    )(rays_o, rays_d)


_TILE = 128
_TPTS = _TILE // 8


_STILE = 4
_SROWS = _STILE * _TILE
_SPTS = _STILE * _TPTS


def _sc_pool(pk, table):
    info = plsc.get_sparse_core_info()
    nc, ns = info.num_cores, info.num_subcores
    nw = nc * ns
    rows_pw = NP8 // nw
    pts_pw = P // nw
    nsup = rows_pw // _SROWS
    mesh = plsc.VectorSubcoreMesh(core_axis_name="c", subcore_axis_name="s")

    @functools.partial(
        pl.kernel,
        mesh=mesh,
        compiler_params=pltpu.CompilerParams(use_tc_tiling_on_sc=False,
                                             needs_layout_passes=False),
        out_type=jax.ShapeDtypeStruct((P, NUM_ATOMS), jnp.float32),
        scratch_types=[
            pltpu.VMEM((2, 2, _SROWS), jnp.int32),
            pltpu.VMEM((2, _SROWS, NUM_ATOMS), jnp.float32),
            pltpu.VMEM((2, _SPTS, NUM_ATOMS), jnp.float32),
            pltpu.SemaphoreType.DMA,
            pltpu.SemaphoreType.DMA,
            pltpu.SemaphoreType.DMA,
            pltpu.SemaphoreType.DMA,
            pltpu.SemaphoreType.DMA,
        ],
    )
    def k(pk_hbm, table_hbm, out_hbm, pk_v, rows_v, code_v,
          psem0, psem1, gsem, ssem0, ssem1):
        wid = lax.axis_index("s") * nc + lax.axis_index("c")
        sbase = wid * nsup
        pbase = wid * pts_pw
        psems = (psem0, psem1)
        ssems = (ssem0, ssem1)

        def stage_desc(sup, par):
            return pltpu.make_async_copy(pk_hbm.at[sbase + sup],
                                         pk_v.at[par], psems[par])

        def fire_gathers(par):
            for j in range(_STILE):
                pltpu.async_copy(
                    table_hbm.at[pk_v.at[par, 0, pl.ds(j * _TILE, _TILE)]],
                    rows_v.at[par, pl.ds(j * _TILE, _TILE)], gsem)

        def drain_gathers(par):
            for j in range(_STILE):
                pltpu.make_async_copy(
                    table_hbm.at[pk_v.at[par, 0, pl.ds(j * _TILE, _TILE)]],
                    rows_v.at[par, pl.ds(j * _TILE, _TILE)], gsem).wait()

        def pool(par):
            for jj in range(_SPTS // 2):
                wchunk = plsc.bitcast(pk_v[par, 1, pl.ds(16 * jj, 16)],
                                      jnp.float32)
                for pi in range(2):
                    i = 2 * jj + pi
                    ws = [wchunk[8 * pi + c] for c in range(8)]
                    for h in range(NUM_ATOMS // 16):
                        acc = ws[0] * rows_v[par, 8 * i, pl.ds(16 * h, 16)]
                        for c in range(1, 8):
                            acc = acc + ws[c] * rows_v[par, 8 * i + c,
                                                       pl.ds(16 * h, 16)]
                        code_v[par, i, pl.ds(16 * h, 16)] = acc

        def store_desc(sup, par):
            poff = pbase + sup * _SPTS
            return pltpu.make_async_copy(
                code_v.at[par], out_hbm.at[pl.ds(poff, _SPTS)], ssems[par])

        stage_desc(0, 0).start()
        stage_desc(1, 1).start()
        stage_desc(0, 0).wait()
        fire_gathers(0)

        def body(g, carry):
            for par in range(2):
                sup = 2 * g + par
                other = 1 - par
                drain_gathers(par)
                if par == 0:
                    stage_desc(sup + 1, other).wait()
                    fire_gathers(other)
                else:
                    @pl.when(g < nsup // 2 - 1)
                    def _():
                        stage_desc(sup + 1, other).wait()
                        fire_gathers(other)
                @pl.when(g >= 1)
                def _():
                    store_desc(0, par).wait()
                pool(par)

                @pl.when(g < nsup // 2 - 1)
                def _():
                    stage_desc(sup + 2, par).start()
                store_desc(sup, par).start()
            return carry

        lax.fori_loop(0, nsup // 2, body, 0)
        store_desc(0, 0).wait()
        store_desc(0, 1).wait()

    return k(pk, table)


_REND_RB = 8


def _render_body(code_ref, o_ref, d_ref, m_ref, rgb_ref, alpha_ref, depth_ref):
    o = o_ref[...]
    d = d_ref[...]
    off_in = jnp.minimum((RADIUS - o) / d, (-RADIUS - o) / d)
    start = jnp.max(off_in, axis=1, keepdims=True)
    steps = lax.broadcasted_iota(jnp.int32, (1, NSAMP), 1).astype(jnp.float32)
    t = start + steps * STEP

    fine_w = []
    inb = None
    for a in range(3):
        pa = o[:, a:a + 1] + t * d[:, a:a + 1]
        in_a = (pa > -RADIUS) & (pa < RADIUS)
        inb = in_a if inb is None else (inb & in_a)
        pn = jnp.clip((pa + RADIUS) / (2.0 * RADIUS), 0.0, 1.0)
        xc = pn * COARSE
        local = xc - jnp.floor(xc)
        xf = local * FINE - 0.5
        j0f = jnp.floor(xf)
        ff = xf - j0f
        a1 = jnp.where(j0f < -0.5, 0.0, jnp.where(j0f > 0.5, 1.0, ff))
        fine_w.append((1.0 - a1, a1))

    code2 = code_ref[...]
    m = m_ref[...]
    data3 = jnp.zeros((_REND_RB, NSAMP, DATA_DIM), jnp.float32)
    f = 0
    for jx in (0, 1):
        for jy in (0, 1):
            for jz in (0, 1):
                wf = fine_w[0][jx] * fine_w[1][jy] * fine_w[2][jz]
                mf = m[f * NUM_ATOMS:(f + 1) * NUM_ATOMS, :]
                tf = jnp.dot(code2, mf, preferred_element_type=jnp.float32)
                data3 = data3 + wf[:, :, None] * tf.reshape(_REND_RB, NSAMP, DATA_DIM)
                f += 1

    svalid = lax.broadcasted_iota(jnp.int32, (1, NSAMP), 1) < NREAL
    mask = inb & svalid
    sig_raw = data3[:, :, DATA_DIM - 1]
    sigma = jnp.where(mask, jnp.maximum(sig_raw, 0.0), 0.0)
    dn = jnp.sqrt(jnp.sum(d * d, axis=1, keepdims=True))
    t_next = start + (steps + 1.0) * STEP
    deltas = (t_next - t) * dn
    x = sigma * deltas
    alpha = 1.0 - jnp.exp(-x)
    tau = jnp.log((1.0 - alpha) + 1e-10)
    ri = lax.broadcasted_iota(jnp.int32, (NSAMP, NSAMP), 0)
    ci = lax.broadcasted_iota(jnp.int32, (NSAMP, NSAMP), 1)
    lt = (ri < ci).astype(jnp.float32)
    texc = jnp.exp(jnp.dot(tau, lt, preferred_element_type=jnp.float32))
    w_al = alpha * texc
    acc_w = jnp.sum(w_al, axis=1, keepdims=True)

    dx_ = d[:, 0:1]
    dy_ = d[:, 1:2]
    dz_ = d[:, 2:3]
    sh = [
        jnp.full_like(dx_, _C0),
        -_C1 * dy_,
        _C1 * dz_,
        -_C1 * dx_,
        _C2 * dx_ * dy_,
        -_C2 * dy_ * dz_,
        _C3 * (2.0 * dz_ * dz_ - dx_ * dx_ - dy_ * dy_),
        -_C2 * dx_ * dz_,
        _C4 * (dx_ * dx_ - dy_ * dy_),
    ]
    outs = []
    for kk in range(3):
        rgb_k = jnp.zeros((_REND_RB, NSAMP), jnp.float32)
        for s in range(SH_DIM):
            rgb_k = rgb_k + data3[:, :, kk * SH_DIM + s] * sh[s]
        csig = 1.0 / (1.0 + jnp.exp(-rgb_k))
        outs.append(jnp.sum(w_al * csig, axis=1, keepdims=True) + 1.0 - acc_w)
    rgb_ref[...] = jnp.concatenate(outs, axis=1)
    alpha_ref[...] = alpha
    depth_ref[...] = jnp.sum(w_al * t, axis=1, keepdims=True)


def _render_call(code, rays_o, rays_d, m2):
    return pl.pallas_call(
        _render_body,
        grid=(BATCH // _REND_RB,),
        in_specs=[
            pl.BlockSpec((_REND_RB * NSAMP, NUM_ATOMS), lambda i: (i, 0)),
            pl.BlockSpec((_REND_RB, 3), lambda i: (i, 0)),
            pl.BlockSpec((_REND_RB, 3), lambda i: (i, 0)),
            pl.BlockSpec((8 * NUM_ATOMS, DATA_DIM), lambda i: (0, 0)),
        ],
        out_specs=[
            pl.BlockSpec((_REND_RB, 3), lambda i: (i, 0)),
            pl.BlockSpec((_REND_RB, NSAMP), lambda i: (i, 0)),
            pl.BlockSpec((_REND_RB, 1), lambda i: (i, 0)),
        ],
        out_shape=[
            jax.ShapeDtypeStruct((BATCH, 3), jnp.float32),
            jax.ShapeDtypeStruct((BATCH, NSAMP), jnp.float32),
            jax.ShapeDtypeStruct((BATCH, 1), jnp.float32),
        ],
    )(code, rays_o, rays_d, m2)


def kernel(rays_o, rays_d, grid_id, grid, atoms):
    del grid_id
    idx8, wc8 = _prep_call(rays_o, rays_d)
    idx_pm = jnp.transpose(idx8, (1, 2, 0)).reshape(-1, 1, _SROWS)
    wc_pm = lax.bitcast_convert_type(
        jnp.transpose(wc8, (1, 2, 0)).reshape(-1, 1, _SROWS), jnp.int32)
    pk = jnp.concatenate([idx_pm, wc_pm], axis=1)
    code = _sc_pool(pk, grid)
    m2 = atoms.reshape(8 * NUM_ATOMS, DATA_DIM)
    rgb, alpha_full, depth2 = _render_call(code, rays_o, rays_d, m2)
    return (rgb, alpha_full[:, :NREAL], depth2.reshape(BATCH))
